# baseline (speedup 1.0000x reference)
import jax
import jax.numpy as jnp
from jax import lax
from jax.experimental import pallas as pl
from jax.experimental.pallas import tpu as pltpu
from jax.experimental.pallas import tpu_sc as plsc

B = 4096
M = 50
EMB = 64
H1 = 64
H2 = 32
N_ROWS = 1000000

NC = 2
NS = 16
NW = NC * NS
U_PER_W = B // NW
R_PER_W = U_PER_W * M
CHUNK = 128
N_CHUNKS = R_PER_W // CHUNK

PCOL = 4096
PGRID = (N_ROWS + PCOL - 1) // PCOL

GB = 64
ROWS = GB * M

LANES = 16


def _sc_users_body(users_hbm, umlpT_hbm, umfT_hbm,
                   uml_out, umf_out,
                   uidx_v, blk_a, blk_b, stage_a, stage_b, sem_a, sem_b):
    wid = lax.axis_index("s") * NC + lax.axis_index("c")
    ubase = wid * U_PER_W
    pltpu.sync_copy(users_hbm.at[pl.ds(ubase, U_PER_W)], uidx_v)
    lane_iota = lax.broadcasted_iota(jnp.int32, (LANES,), 0)

    def group(gi, carry):
        gvec = uidx_v[pl.ds(gi * LANES, LANES)]
        for j in range(LANES):
            i = jnp.sum(jnp.where(lane_iota == j, gvec, 0))
            g = pl.multiple_of((i // 128) * 128, 128)
            c = i - g
            cp_a = pltpu.async_copy(
                umlpT_hbm.at[:, pl.ds(g, 128)], blk_a, sem_a)
            cp_b = pltpu.async_copy(
                umfT_hbm.at[:, pl.ds(g, 128)], blk_b, sem_b)
            cidx = jnp.full((LANES,), c, jnp.int32)
            k = gi * LANES + j
            cp_a.wait()
            for t in range(EMB // LANES):
                ridx = lane_iota + (LANES * t)
                stage_a[k, pl.ds(LANES * t, LANES)] = plsc.load_gather(
                    blk_a, [ridx, cidx])
            cp_b.wait()
            for t in range(EMB // LANES):
                ridx = lane_iota + (LANES * t)
                stage_b[k, pl.ds(LANES * t, LANES)] = plsc.load_gather(
                    blk_b, [ridx, cidx])
        return carry

    lax.fori_loop(0, U_PER_W // LANES, group, 0)
    pltpu.sync_copy(stage_a, uml_out.at[pl.ds(ubase, U_PER_W)])
    pltpu.sync_copy(stage_b, umf_out.at[pl.ds(ubase, U_PER_W)])


_sc_users = pl.kernel(
    _sc_users_body,
    out_type=(
        jax.ShapeDtypeStruct((B, EMB), jnp.float32),
        jax.ShapeDtypeStruct((B, EMB), jnp.float32),
    ),
    mesh=plsc.VectorSubcoreMesh(
        core_axis_name="c", subcore_axis_name="s",
        num_cores=NC, num_subcores=NS),
    scratch_types=[
        pltpu.VMEM((U_PER_W,), jnp.int32),
        pltpu.VMEM((EMB, 128), jnp.float32),
        pltpu.VMEM((EMB, 128), jnp.float32),
        pltpu.VMEM((U_PER_W, EMB), jnp.float32),
        pltpu.VMEM((U_PER_W, EMB), jnp.float32),
        pltpu.SemaphoreType.DMA,
        pltpu.SemaphoreType.DMA,
    ],
    compiler_params=pltpu.CompilerParams(needs_layout_passes=False),
)


_DN_T = (((0,), (0,)), ((), ()))


def _pack_body(imlp_ref, imf_ref, ident_ref, pi_ref):
    ident = ident_ref[...]
    pi_ref[:, 0:EMB] = lax.dot_general(
        imlp_ref[...], ident, _DN_T, preferred_element_type=jnp.float32)
    pi_ref[:, EMB:2 * EMB] = lax.dot_general(
        imf_ref[...], ident, _DN_T, preferred_element_type=jnp.float32)


_pack = pl.pallas_call(
    _pack_body,
    grid=(PGRID,),
    in_specs=[
        pl.BlockSpec((EMB, PCOL), lambda i: (0, i)),
        pl.BlockSpec((EMB, PCOL), lambda i: (0, i)),
        pl.BlockSpec((EMB, EMB), lambda i: (0, 0)),
    ],
    out_specs=pl.BlockSpec((PCOL, 2 * EMB), lambda i: (i, 0)),
    out_shape=jax.ShapeDtypeStruct((N_ROWS, 2 * EMB), jnp.float32),
)


def _sc_items_body(items_hbm, pi_hbm, v_out, iidx_v, vrows_v, sem):
    wid = lax.axis_index("s") * NC + lax.axis_index("c")
    rbase = wid * R_PER_W

    def chunk(c, carry):
        r0 = rbase + c * CHUNK
        pltpu.sync_copy(items_hbm.at[pl.ds(r0, CHUNK)], iidx_v)
        pltpu.async_copy(pi_hbm.at[iidx_v], vrows_v, sem).wait()
        pltpu.sync_copy(vrows_v, v_out.at[pl.ds(r0, CHUNK)])
        return carry

    lax.fori_loop(0, N_CHUNKS, chunk, 0)


_sc_items = pl.kernel(
    _sc_items_body,
    out_type=jax.ShapeDtypeStruct((B * M, 2 * EMB), jnp.float32),
    mesh=plsc.VectorSubcoreMesh(
        core_axis_name="c", subcore_axis_name="s",
        num_cores=NC, num_subcores=NS),
    scratch_types=[
        pltpu.VMEM((CHUNK,), jnp.int32),
        pltpu.VMEM((CHUNK, 2 * EMB), jnp.float32),
        pltpu.SemaphoreType.DMA,
    ],
)


def _tc_body(v_ref, uml_ref, umf_ref, s_ref, w1u_ref, b1_ref,
             w1v_ref, w2_ref, b2_ref, out_mlp_ref, out_mf_ref):
    vp = v_ref[...]
    up = jnp.dot(uml_ref[...], w1u_ref[...],
                 preferred_element_type=jnp.float32) + b1_ref[...]
    u_cat = jnp.concatenate([up, umf_ref[...]], axis=1)
    z = jnp.dot(s_ref[...], u_cat,
                preferred_element_type=jnp.float32)
    h = jnp.dot(vp, w1v_ref[...],
                preferred_element_type=jnp.float32) + z[:, 0:H1]
    h = jnp.maximum(h, 0.0)
    out_mlp_ref[...] = jnp.dot(h, w2_ref[...],
                               preferred_element_type=jnp.float32) + b2_ref[...]
    out_mf_ref[...] = z[:, EMB:2 * EMB] * vp[:, EMB:2 * EMB]


_tc_dense = pl.pallas_call(
    _tc_body,
    grid=(B // GB,),
    in_specs=[
        pl.BlockSpec((ROWS, 2 * EMB), lambda i: (i, 0)),
        pl.BlockSpec((GB, EMB), lambda i: (i, 0)),
        pl.BlockSpec((GB, EMB), lambda i: (i, 0)),
        pl.BlockSpec((ROWS, GB), lambda i: (0, 0)),
        pl.BlockSpec((EMB, H1), lambda i: (0, 0)),
        pl.BlockSpec((1, H1), lambda i: (0, 0)),
        pl.BlockSpec((2 * EMB, H1), lambda i: (0, 0)),
        pl.BlockSpec((H1, H2), lambda i: (0, 0)),
        pl.BlockSpec((1, H2), lambda i: (0, 0)),
    ],
    out_specs=[
        pl.BlockSpec((ROWS, H2), lambda i: (i, 0)),
        pl.BlockSpec((ROWS, EMB), lambda i: (i, 0)),
    ],
    out_shape=[
        jax.ShapeDtypeStruct((B * M, H2), jnp.float32),
        jax.ShapeDtypeStruct((B * M, EMB), jnp.float32),
    ],
)


def kernel(users, items, user_mlp_table, user_mf_table,
           item_mlp_table, item_mf_table, W1, b1, W2, b2):
    users = users.astype(jnp.int32)
    items_flat = items.reshape(-1).astype(jnp.int32)

    uml, umf = _sc_users(users, user_mlp_table.T, user_mf_table.T)

    packed_items = _pack(item_mlp_table.T, item_mf_table.T,
                         jnp.eye(EMB, dtype=jnp.float32))

    v_packed = _sc_items(items_flat, packed_items)

    s = (lax.broadcasted_iota(jnp.int32, (ROWS, GB), 0) // M
         == lax.broadcasted_iota(jnp.int32, (ROWS, GB), 1)).astype(jnp.float32)
    w1v_pad = jnp.concatenate(
        [W1[EMB:], jnp.zeros((EMB, H1), jnp.float32)], axis=0)

    out_mlp, out_mf = _tc_dense(
        v_packed, uml, umf,
        s, W1[:EMB], b1.reshape(1, H1), w1v_pad, W2, b2.reshape(1, H2))

    return out_mlp.reshape(B, M, H2), out_mf.reshape(B, M, EMB)


# speedup vs baseline: 1.1325x; 1.1325x over previous
import jax
import jax.numpy as jnp
from jax import lax
from jax.experimental import pallas as pl
from jax.experimental.pallas import tpu as pltpu
from jax.experimental.pallas import tpu_sc as plsc

B = 4096
M = 50
EMB = 64
H1 = 64
H2 = 32
N_ROWS = 1000000

NC = 2
NS = 16
NW = NC * NS
U_PER_W = B // NW
R_PER_W = U_PER_W * M
CHUNK = 128
N_CHUNKS = R_PER_W // CHUNK

PCOL = 4096
PGRID = (N_ROWS + PCOL - 1) // PCOL

GB = 64
ROWS = GB * M

LANES = 16


def _sc_users_body(users_hbm, umlpT_hbm, umfT_hbm,
                   uml_out, umf_out,
                   uidx_v, blk_a, blk_b, row_v, sem_a, sem_b):
    wid = lax.axis_index("s") * NC + lax.axis_index("c")
    ubase = wid * U_PER_W
    pltpu.sync_copy(users_hbm.at[pl.ds(ubase, U_PER_W)], uidx_v)
    lane_iota = lax.broadcasted_iota(jnp.int32, (LANES,), 0)

    def group(gi, carry):
        gvec = uidx_v[pl.ds(gi * LANES, LANES)]
        for j in range(LANES):
            i = jnp.sum(jnp.where(lane_iota == j, gvec, 0))
            g = pl.multiple_of((i // 128) * 128, 128)
            c = i - g
            cp_a = pltpu.async_copy(
                umlpT_hbm.at[:, pl.ds(g, 128)], blk_a, sem_a)
            cp_b = pltpu.async_copy(
                umfT_hbm.at[:, pl.ds(g, 128)], blk_b, sem_b)
            cidx = jnp.full((LANES,), c, jnp.int32)
            k = ubase + gi * LANES + j
            cp_a.wait()
            for t in range(EMB // LANES):
                ridx = lane_iota + (LANES * t)
                row_v[pl.ds(LANES * t, LANES)] = plsc.load_gather(
                    blk_a, [ridx, cidx])
            pltpu.sync_copy(row_v, uml_out.at[pl.ds(k * EMB, EMB)])
            cp_b.wait()
            for t in range(EMB // LANES):
                ridx = lane_iota + (LANES * t)
                row_v[pl.ds(LANES * t, LANES)] = plsc.load_gather(
                    blk_b, [ridx, cidx])
            pltpu.sync_copy(row_v, umf_out.at[pl.ds(k * EMB, EMB)])
        return carry

    lax.fori_loop(0, U_PER_W // LANES, group, 0)


_sc_users = pl.kernel(
    _sc_users_body,
    out_type=(
        jax.ShapeDtypeStruct((B * EMB,), jnp.float32),
        jax.ShapeDtypeStruct((B * EMB,), jnp.float32),
    ),
    mesh=plsc.VectorSubcoreMesh(
        core_axis_name="c", subcore_axis_name="s",
        num_cores=NC, num_subcores=NS),
    scratch_types=[
        pltpu.VMEM((U_PER_W,), jnp.int32),
        pltpu.VMEM((EMB, 128), jnp.float32),
        pltpu.VMEM((EMB, 128), jnp.float32),
        pltpu.VMEM((EMB,), jnp.float32),
        pltpu.SemaphoreType.DMA,
        pltpu.SemaphoreType.DMA,
    ],
    compiler_params=pltpu.CompilerParams(needs_layout_passes=False),
)


_DN_T = (((0,), (0,)), ((), ()))


def _pack_body(imlp_ref, imf_ref, ident_ref, pi_ref):
    ident = ident_ref[...]
    pi_ref[:, 0:EMB] = lax.dot_general(
        imlp_ref[...], ident, _DN_T, preferred_element_type=jnp.float32)
    pi_ref[:, EMB:2 * EMB] = lax.dot_general(
        imf_ref[...], ident, _DN_T, preferred_element_type=jnp.float32)


_pack = pl.pallas_call(
    _pack_body,
    grid=(PGRID,),
    in_specs=[
        pl.BlockSpec((EMB, PCOL), lambda i: (0, i)),
        pl.BlockSpec((EMB, PCOL), lambda i: (0, i)),
        pl.BlockSpec((EMB, EMB), lambda i: (0, 0)),
    ],
    out_specs=pl.BlockSpec((PCOL, 2 * EMB), lambda i: (i, 0)),
    out_shape=jax.ShapeDtypeStruct((N_ROWS, 2 * EMB), jnp.float32),
)


def _sc_items_body(items_hbm, pi_hbm, v_out, iidx_v, vrows_v, sem):
    wid = lax.axis_index("s") * NC + lax.axis_index("c")
    rbase = wid * R_PER_W

    def chunk(c, carry):
        r0 = rbase + c * CHUNK
        pltpu.sync_copy(items_hbm.at[pl.ds(r0, CHUNK)], iidx_v)
        pltpu.async_copy(pi_hbm.at[iidx_v], vrows_v, sem).wait()
        pltpu.sync_copy(vrows_v, v_out.at[pl.ds(r0, CHUNK)])
        return carry

    lax.fori_loop(0, N_CHUNKS, chunk, 0)


_sc_items = pl.kernel(
    _sc_items_body,
    out_type=jax.ShapeDtypeStruct((B * M, 2 * EMB), jnp.float32),
    mesh=plsc.VectorSubcoreMesh(
        core_axis_name="c", subcore_axis_name="s",
        num_cores=NC, num_subcores=NS),
    scratch_types=[
        pltpu.VMEM((CHUNK,), jnp.int32),
        pltpu.VMEM((CHUNK, 2 * EMB), jnp.float32),
        pltpu.SemaphoreType.DMA,
    ],
)


def _tc_body(v_ref, uml_ref, umf_ref, s_ref, w1u_ref, b1_ref,
             w1v_ref, w2_ref, b2_ref, out_mlp_ref, out_mf_ref):
    vp = v_ref[...]
    up = jnp.dot(uml_ref[...], w1u_ref[...],
                 preferred_element_type=jnp.float32) + b1_ref[...]
    u_cat = jnp.concatenate([up, umf_ref[...]], axis=1)
    z = jnp.dot(s_ref[...], u_cat,
                preferred_element_type=jnp.float32)
    h = jnp.dot(vp, w1v_ref[...],
                preferred_element_type=jnp.float32) + z[:, 0:H1]
    h = jnp.maximum(h, 0.0)
    out_mlp_ref[...] = jnp.dot(h, w2_ref[...],
                               preferred_element_type=jnp.float32) + b2_ref[...]
    out_mf_ref[...] = z[:, EMB:2 * EMB] * vp[:, EMB:2 * EMB]


_tc_dense = pl.pallas_call(
    _tc_body,
    grid=(B // GB,),
    in_specs=[
        pl.BlockSpec((ROWS, 2 * EMB), lambda i: (i, 0)),
        pl.BlockSpec((GB, EMB), lambda i: (i, 0)),
        pl.BlockSpec((GB, EMB), lambda i: (i, 0)),
        pl.BlockSpec((ROWS, GB), lambda i: (0, 0)),
        pl.BlockSpec((EMB, H1), lambda i: (0, 0)),
        pl.BlockSpec((1, H1), lambda i: (0, 0)),
        pl.BlockSpec((2 * EMB, H1), lambda i: (0, 0)),
        pl.BlockSpec((H1, H2), lambda i: (0, 0)),
        pl.BlockSpec((1, H2), lambda i: (0, 0)),
    ],
    out_specs=[
        pl.BlockSpec((ROWS, H2), lambda i: (i, 0)),
        pl.BlockSpec((ROWS, EMB), lambda i: (i, 0)),
    ],
    out_shape=[
        jax.ShapeDtypeStruct((B * M, H2), jnp.float32),
        jax.ShapeDtypeStruct((B * M, EMB), jnp.float32),
    ],
)


def kernel(users, items, user_mlp_table, user_mf_table,
           item_mlp_table, item_mf_table, W1, b1, W2, b2):
    users = users.astype(jnp.int32)
    items_flat = items.reshape(-1).astype(jnp.int32)

    uml_flat, umf_flat = _sc_users(users, user_mlp_table.T, user_mf_table.T)

    packed_items = _pack(item_mlp_table.T, item_mf_table.T,
                         jnp.eye(EMB, dtype=jnp.float32))

    v_packed = _sc_items(items_flat, packed_items)

    s = (lax.broadcasted_iota(jnp.int32, (ROWS, GB), 0) // M
         == lax.broadcasted_iota(jnp.int32, (ROWS, GB), 1)).astype(jnp.float32)
    w1v_pad = jnp.concatenate(
        [W1[EMB:], jnp.zeros((EMB, H1), jnp.float32)], axis=0)

    out_mlp, out_mf = _tc_dense(
        v_packed, uml_flat.reshape(B, EMB), umf_flat.reshape(B, EMB),
        s, W1[:EMB], b1.reshape(1, H1), w1v_pad, W2, b2.reshape(1, H2))

    return out_mlp.reshape(B, M, H2), out_mf.reshape(B, M, EMB)


# speedup vs baseline: 1.1340x; 1.0013x over previous
import jax
import jax.numpy as jnp
from jax import lax
from jax.experimental import pallas as pl
from jax.experimental.pallas import tpu as pltpu
from jax.experimental.pallas import tpu_sc as plsc

B = 4096
M = 50
EMB = 64
H1 = 64
H2 = 32
N_ROWS = 1000000

NC = 2
NS = 16
NW = NC * NS
U_PER_W = B // NW
R_PER_W = U_PER_W * M
CHUNK = 128
N_CHUNKS = R_PER_W // CHUNK

PCOL = 4096
PGRID = (N_ROWS + PCOL - 1) // PCOL

GB = 64
ROWS = GB * M

LANES = 16


def _sc_users_body(users_hbm, umlpT_hbm, umfT_hbm,
                   uml_out, umf_out,
                   uidx_v, blk_a, blk_b, row_v, sem_a, sem_b):
    wid = lax.axis_index("s") * NC + lax.axis_index("c")
    ubase = wid * U_PER_W
    pltpu.sync_copy(users_hbm.at[pl.ds(ubase, U_PER_W)], uidx_v)
    lane_iota = lax.broadcasted_iota(jnp.int32, (LANES,), 0)

    def group(gi, carry):
        gvec = uidx_v[pl.ds(gi * LANES, LANES)]
        for j in range(LANES):
            i = jnp.sum(jnp.where(lane_iota == j, gvec, 0))
            g = pl.multiple_of((i // 128) * 128, 128)
            c = i - g
            cp_a = pltpu.async_copy(
                umlpT_hbm.at[:, pl.ds(g, 128)], blk_a, sem_a)
            cp_b = pltpu.async_copy(
                umfT_hbm.at[:, pl.ds(g, 128)], blk_b, sem_b)
            cidx = jnp.full((LANES,), c, jnp.int32)
            k = ubase + gi * LANES + j
            cp_a.wait()
            for t in range(EMB // LANES):
                ridx = lane_iota + (LANES * t)
                row_v[pl.ds(LANES * t, LANES)] = plsc.load_gather(
                    blk_a, [ridx, cidx])
            pltpu.sync_copy(row_v, uml_out.at[pl.ds(k * EMB, EMB)])
            cp_b.wait()
            for t in range(EMB // LANES):
                ridx = lane_iota + (LANES * t)
                row_v[pl.ds(LANES * t, LANES)] = plsc.load_gather(
                    blk_b, [ridx, cidx])
            pltpu.sync_copy(row_v, umf_out.at[pl.ds(k * EMB, EMB)])
        return carry

    lax.fori_loop(0, U_PER_W // LANES, group, 0)


_sc_users = pl.kernel(
    _sc_users_body,
    out_type=(
        jax.ShapeDtypeStruct((B * EMB,), jnp.float32),
        jax.ShapeDtypeStruct((B * EMB,), jnp.float32),
    ),
    mesh=plsc.VectorSubcoreMesh(
        core_axis_name="c", subcore_axis_name="s",
        num_cores=NC, num_subcores=NS),
    scratch_types=[
        pltpu.VMEM((U_PER_W,), jnp.int32),
        pltpu.VMEM((EMB, 128), jnp.float32),
        pltpu.VMEM((EMB, 128), jnp.float32),
        pltpu.VMEM((EMB,), jnp.float32),
        pltpu.SemaphoreType.DMA,
        pltpu.SemaphoreType.DMA,
    ],
    compiler_params=pltpu.CompilerParams(needs_layout_passes=False),
)


_DN_T = (((0,), (0,)), ((), ()))


def _pack_body(imlp_ref, imf_ref, w1v_ref, ident_ref, pi_ref):
    pi_ref[:, 0:EMB] = lax.dot_general(
        imlp_ref[...], w1v_ref[...], _DN_T, preferred_element_type=jnp.float32)
    pi_ref[:, EMB:2 * EMB] = lax.dot_general(
        imf_ref[...], ident_ref[...], _DN_T,
        preferred_element_type=jnp.float32)


_pack = pl.pallas_call(
    _pack_body,
    grid=(PGRID,),
    in_specs=[
        pl.BlockSpec((EMB, PCOL), lambda i: (0, i)),
        pl.BlockSpec((EMB, PCOL), lambda i: (0, i)),
        pl.BlockSpec((EMB, H1), lambda i: (0, 0)),
        pl.BlockSpec((EMB, EMB), lambda i: (0, 0)),
    ],
    out_specs=pl.BlockSpec((PCOL, 2 * EMB), lambda i: (i, 0)),
    out_shape=jax.ShapeDtypeStruct((N_ROWS, 2 * EMB), jnp.float32),
)


def _sc_items_body(items_hbm, pi_hbm, v_out, iidx_v, vrows_v, sem):
    wid = lax.axis_index("s") * NC + lax.axis_index("c")
    rbase = wid * R_PER_W

    def chunk(c, carry):
        r0 = rbase + c * CHUNK
        pltpu.sync_copy(items_hbm.at[pl.ds(r0, CHUNK)], iidx_v)
        pltpu.async_copy(pi_hbm.at[iidx_v], vrows_v, sem).wait()
        pltpu.sync_copy(vrows_v, v_out.at[pl.ds(r0, CHUNK)])
        return carry

    lax.fori_loop(0, N_CHUNKS, chunk, 0)


_sc_items = pl.kernel(
    _sc_items_body,
    out_type=jax.ShapeDtypeStruct((B * M, 2 * EMB), jnp.float32),
    mesh=plsc.VectorSubcoreMesh(
        core_axis_name="c", subcore_axis_name="s",
        num_cores=NC, num_subcores=NS),
    scratch_types=[
        pltpu.VMEM((CHUNK,), jnp.int32),
        pltpu.VMEM((CHUNK, 2 * EMB), jnp.float32),
        pltpu.SemaphoreType.DMA,
    ],
)


def _tc_body(v_ref, uml_ref, umf_ref, s_ref, w1u_ref, b1_ref,
             w2_ref, b2_ref, out_mlp_ref, out_mf_ref):
    vp = v_ref[...]
    up = jnp.dot(uml_ref[...], w1u_ref[...],
                 preferred_element_type=jnp.float32) + b1_ref[...]
    u_cat = jnp.concatenate([up, umf_ref[...]], axis=1)
    z = jnp.dot(s_ref[...], u_cat,
                preferred_element_type=jnp.float32)
    zv = z + vp
    h = jnp.maximum(zv[:, 0:H1], 0.0)
    out_mlp_ref[...] = jnp.dot(h, w2_ref[...],
                               preferred_element_type=jnp.float32) + b2_ref[...]
    out_mf_ref[...] = z[:, EMB:2 * EMB] * vp[:, EMB:2 * EMB]


_tc_dense = pl.pallas_call(
    _tc_body,
    grid=(B // GB,),
    in_specs=[
        pl.BlockSpec((ROWS, 2 * EMB), lambda i: (i, 0)),
        pl.BlockSpec((GB, EMB), lambda i: (i, 0)),
        pl.BlockSpec((GB, EMB), lambda i: (i, 0)),
        pl.BlockSpec((ROWS, GB), lambda i: (0, 0)),
        pl.BlockSpec((EMB, H1), lambda i: (0, 0)),
        pl.BlockSpec((1, H1), lambda i: (0, 0)),
        pl.BlockSpec((H1, H2), lambda i: (0, 0)),
        pl.BlockSpec((1, H2), lambda i: (0, 0)),
    ],
    out_specs=[
        pl.BlockSpec((ROWS, H2), lambda i: (i, 0)),
        pl.BlockSpec((ROWS, EMB), lambda i: (i, 0)),
    ],
    out_shape=[
        jax.ShapeDtypeStruct((B * M, H2), jnp.float32),
        jax.ShapeDtypeStruct((B * M, EMB), jnp.float32),
    ],
)


def kernel(users, items, user_mlp_table, user_mf_table,
           item_mlp_table, item_mf_table, W1, b1, W2, b2):
    users = users.astype(jnp.int32)
    items_flat = items.reshape(-1).astype(jnp.int32)

    uml_flat, umf_flat = _sc_users(users, user_mlp_table.T, user_mf_table.T)

    packed_items = _pack(item_mlp_table.T, item_mf_table.T,
                         W1[EMB:], jnp.eye(EMB, dtype=jnp.float32))

    v_packed = _sc_items(items_flat, packed_items)

    s = (lax.broadcasted_iota(jnp.int32, (ROWS, GB), 0) // M
         == lax.broadcasted_iota(jnp.int32, (ROWS, GB), 1)).astype(jnp.float32)

    out_mlp, out_mf = _tc_dense(
        v_packed, uml_flat.reshape(B, EMB), umf_flat.reshape(B, EMB),
        s, W1[:EMB], b1.reshape(1, H1), W2, b2.reshape(1, H2))

    return out_mlp.reshape(B, M, H2), out_mf.reshape(B, M, EMB)


# speedup vs baseline: 1.1618x; 1.0245x over previous
import jax
import jax.numpy as jnp
from jax import lax
from jax.experimental import pallas as pl
from jax.experimental.pallas import tpu as pltpu
from jax.experimental.pallas import tpu_sc as plsc

B = 4096
M = 50
EMB = 64
H1 = 64
H2 = 32
N_ROWS = 1000000

NC = 2
NS = 16
NW = NC * NS
U_PER_W = B // NW
R_PER_W = U_PER_W * M
CHUNK = 128
N_CHUNKS = R_PER_W // CHUNK

PCOL = 4096
PGRID = (N_ROWS + PCOL - 1) // PCOL

GB = 64
ROWS = GB * M

LANES = 16


def _sc_users_body(users_hbm, umlpT_hbm, umfT_hbm,
                   uml_out, umf_out,
                   uidx_v, blk_a, blk_b, row_v, sem_a, sem_b):
    wid = lax.axis_index("s") * NC + lax.axis_index("c")
    ubase = wid * U_PER_W
    pltpu.sync_copy(users_hbm.at[pl.ds(ubase, U_PER_W)], uidx_v)
    lane_iota = lax.broadcasted_iota(jnp.int32, (LANES,), 0)

    def group(gi, carry):
        gvec = uidx_v[pl.ds(gi * LANES, LANES)]
        for j in range(LANES):
            i = jnp.sum(jnp.where(lane_iota == j, gvec, 0))
            g = pl.multiple_of((i // 128) * 128, 128)
            c = i - g
            cp_a = pltpu.async_copy(
                umlpT_hbm.at[:, pl.ds(g, 128)], blk_a, sem_a)
            cp_b = pltpu.async_copy(
                umfT_hbm.at[:, pl.ds(g, 128)], blk_b, sem_b)
            cidx = jnp.full((LANES,), c, jnp.int32)
            k = ubase + gi * LANES + j
            cp_a.wait()
            for t in range(EMB // LANES):
                ridx = lane_iota + (LANES * t)
                row_v[pl.ds(LANES * t, LANES)] = plsc.load_gather(
                    blk_a, [ridx, cidx])
            pltpu.sync_copy(row_v, uml_out.at[pl.ds(k * EMB, EMB)])
            cp_b.wait()
            for t in range(EMB // LANES):
                ridx = lane_iota + (LANES * t)
                row_v[pl.ds(LANES * t, LANES)] = plsc.load_gather(
                    blk_b, [ridx, cidx])
            pltpu.sync_copy(row_v, umf_out.at[pl.ds(k * EMB, EMB)])
        return carry

    lax.fori_loop(0, U_PER_W // LANES, group, 0)


_sc_users = pl.kernel(
    _sc_users_body,
    out_type=(
        jax.ShapeDtypeStruct((B * EMB,), jnp.float32),
        jax.ShapeDtypeStruct((B * EMB,), jnp.float32),
    ),
    mesh=plsc.VectorSubcoreMesh(
        core_axis_name="c", subcore_axis_name="s",
        num_cores=NC, num_subcores=NS),
    scratch_types=[
        pltpu.VMEM((U_PER_W,), jnp.int32),
        pltpu.VMEM((EMB, 128), jnp.float32),
        pltpu.VMEM((EMB, 128), jnp.float32),
        pltpu.VMEM((EMB,), jnp.float32),
        pltpu.SemaphoreType.DMA,
        pltpu.SemaphoreType.DMA,
    ],
    compiler_params=pltpu.CompilerParams(needs_layout_passes=False),
)


_DN_T = (((0,), (0,)), ((), ()))


def _pack_body(imlp_ref, imf_ref, w1v_ref, ident_ref, pi_ref):
    pi_ref[:, 0:EMB] = lax.dot_general(
        imlp_ref[...], w1v_ref[...], _DN_T, preferred_element_type=jnp.float32)
    pi_ref[:, EMB:2 * EMB] = lax.dot_general(
        imf_ref[...], ident_ref[...], _DN_T,
        preferred_element_type=jnp.float32)


_pack = pl.pallas_call(
    _pack_body,
    grid=(PGRID,),
    in_specs=[
        pl.BlockSpec((EMB, PCOL), lambda i: (0, i)),
        pl.BlockSpec((EMB, PCOL), lambda i: (0, i)),
        pl.BlockSpec((EMB, H1), lambda i: (0, 0)),
        pl.BlockSpec((EMB, EMB), lambda i: (0, 0)),
    ],
    out_specs=pl.BlockSpec((PCOL, 2 * EMB), lambda i: (i, 0)),
    out_shape=jax.ShapeDtypeStruct((N_ROWS, 2 * EMB), jnp.float32),
)


def _sc_items_body(items_hbm, pi_hbm, v_out,
                   iidx_a, iidx_b, vrows_a, vrows_b, sem_a, sem_b):
    wid = lax.axis_index("s") * NC + lax.axis_index("c")
    rbase = wid * R_PER_W

    def chunk2(p, carry):
        r0 = rbase + p * (2 * CHUNK)
        r1 = r0 + CHUNK
        pltpu.sync_copy(items_hbm.at[pl.ds(r0, CHUNK)], iidx_a)
        pltpu.sync_copy(items_hbm.at[pl.ds(r1, CHUNK)], iidx_b)
        cp_a = pltpu.async_copy(pi_hbm.at[iidx_a], vrows_a, sem_a)
        cp_b = pltpu.async_copy(pi_hbm.at[iidx_b], vrows_b, sem_b)
        cp_a.wait()
        pltpu.sync_copy(vrows_a, v_out.at[pl.ds(r0, CHUNK)])
        cp_b.wait()
        pltpu.sync_copy(vrows_b, v_out.at[pl.ds(r1, CHUNK)])
        return carry

    lax.fori_loop(0, N_CHUNKS // 2, chunk2, 0)


_sc_items = pl.kernel(
    _sc_items_body,
    out_type=jax.ShapeDtypeStruct((B * M, 2 * EMB), jnp.float32),
    mesh=plsc.VectorSubcoreMesh(
        core_axis_name="c", subcore_axis_name="s",
        num_cores=NC, num_subcores=NS),
    scratch_types=[
        pltpu.VMEM((CHUNK,), jnp.int32),
        pltpu.VMEM((CHUNK,), jnp.int32),
        pltpu.VMEM((CHUNK, 2 * EMB), jnp.float32),
        pltpu.VMEM((CHUNK, 2 * EMB), jnp.float32),
        pltpu.SemaphoreType.DMA,
        pltpu.SemaphoreType.DMA,
    ],
)


def _tc_body(v_ref, uml_ref, umf_ref, s_ref, w1u_ref, b1_ref,
             w2_ref, b2_ref, out_mlp_ref, out_mf_ref):
    vp = v_ref[...]
    up = jnp.dot(uml_ref[...], w1u_ref[...],
                 preferred_element_type=jnp.float32) + b1_ref[...]
    u_cat = jnp.concatenate([up, umf_ref[...]], axis=1)
    z = jnp.dot(s_ref[...], u_cat,
                preferred_element_type=jnp.float32)
    zv = z + vp
    h = jnp.maximum(zv[:, 0:H1], 0.0)
    out_mlp_ref[...] = jnp.dot(h, w2_ref[...],
                               preferred_element_type=jnp.float32) + b2_ref[...]
    out_mf_ref[...] = z[:, EMB:2 * EMB] * vp[:, EMB:2 * EMB]


_tc_dense = pl.pallas_call(
    _tc_body,
    grid=(B // GB,),
    in_specs=[
        pl.BlockSpec((ROWS, 2 * EMB), lambda i: (i, 0)),
        pl.BlockSpec((GB, EMB), lambda i: (i, 0)),
        pl.BlockSpec((GB, EMB), lambda i: (i, 0)),
        pl.BlockSpec((ROWS, GB), lambda i: (0, 0)),
        pl.BlockSpec((EMB, H1), lambda i: (0, 0)),
        pl.BlockSpec((1, H1), lambda i: (0, 0)),
        pl.BlockSpec((H1, H2), lambda i: (0, 0)),
        pl.BlockSpec((1, H2), lambda i: (0, 0)),
    ],
    out_specs=[
        pl.BlockSpec((ROWS, H2), lambda i: (i, 0)),
        pl.BlockSpec((ROWS, EMB), lambda i: (i, 0)),
    ],
    out_shape=[
        jax.ShapeDtypeStruct((B * M, H2), jnp.float32),
        jax.ShapeDtypeStruct((B * M, EMB), jnp.float32),
    ],
)


def kernel(users, items, user_mlp_table, user_mf_table,
           item_mlp_table, item_mf_table, W1, b1, W2, b2):
    users = users.astype(jnp.int32)
    items_flat = items.reshape(-1).astype(jnp.int32)

    uml_flat, umf_flat = _sc_users(users, user_mlp_table.T, user_mf_table.T)

    packed_items = _pack(item_mlp_table.T, item_mf_table.T,
                         W1[EMB:], jnp.eye(EMB, dtype=jnp.float32))

    v_packed = _sc_items(items_flat, packed_items)

    s = (lax.broadcasted_iota(jnp.int32, (ROWS, GB), 0) // M
         == lax.broadcasted_iota(jnp.int32, (ROWS, GB), 1)).astype(jnp.float32)

    out_mlp, out_mf = _tc_dense(
        v_packed, uml_flat.reshape(B, EMB), umf_flat.reshape(B, EMB),
        s, W1[:EMB], b1.reshape(1, H1), W2, b2.reshape(1, H2))

    return out_mlp.reshape(B, M, H2), out_mf.reshape(B, M, EMB)
